# baseline (speedup 1.0000x reference)
import jax
import jax.numpy as jnp
from jax import lax
from jax.experimental import pallas as pl
from jax.experimental.pallas import tpu as pltpu
from jax.experimental.pallas import tpu_sc as plsc

_LANES = 16
_NUM_CORES = 2
_NUM_SUBCORES = 16
_NW = _NUM_CORES * _NUM_SUBCORES

_B, _R, _F = 4, 4096, 256
_OF = _F // 2
_ROWS = _B * _R
_ROWS_PER_W = _ROWS // _NW
_NCHUNK = 8
_CROWS = _ROWS_PER_W // _NCHUNK
_VPR = _OF // _LANES
_RUNROLL = 2


def _sel_body(x_hbm, out_hbm, in_v0, in_v1, out_v0, out_v1, in_sem, out_sem):
    in_bufs = (in_v0, in_v1)
    out_bufs = (out_v0, out_v1)
    wid = lax.axis_index("s") * _NUM_CORES + lax.axis_index("c")
    b = wid // (_R // _ROWS_PER_W)
    r0 = (wid % (_R // _ROWS_PER_W)) * _ROWS_PER_W

    iota = lax.iota(jnp.int32, _LANES)
    cols = [iota * 2 + 32 * vj for vj in range(_VPR)]

    def start_in(t):
        return pltpu.async_copy(
            x_hbm.at[b, pl.ds(r0 + t * _CROWS, _CROWS), :],
            in_bufs[t % 2], in_sem)

    def start_out(t):
        return pltpu.async_copy(
            out_bufs[t % 2],
            out_hbm.at[b, pl.ds(r0 + t * _CROWS, _CROWS), :], out_sem)

    copies_in = [start_in(0)]
    copies_out = []
    for t in range(_NCHUNK):
        if t + 1 < _NCHUNK:
            copies_in.append(start_in(t + 1))
        copies_in[t].wait()
        if t >= 2:
            copies_out[t - 2].wait()
        src = in_bufs[t % 2]
        dst = out_bufs[t % 2]

        def body(i, carry):
            dr0 = i * _RUNROLL
            for u in range(_RUNROLL):
                dr = dr0 + u
                row = iota * 0 + dr
                for vj in range(_VPR):
                    dst[dr, pl.ds(vj * _LANES, _LANES)] = (
                        plsc.load_gather(src, [row, cols[vj]]))
            return carry

        lax.fori_loop(0, _CROWS // _RUNROLL, body, 0)
        copies_out.append(start_out(t))
    copies_out[-2].wait()
    copies_out[-1].wait()


_sel = pl.kernel(
    _sel_body,
    out_type=jax.ShapeDtypeStruct((_B, _R, _OF), jnp.float32),
    mesh=plsc.VectorSubcoreMesh(
        core_axis_name="c",
        subcore_axis_name="s",
        num_cores=_NUM_CORES,
        num_subcores=_NUM_SUBCORES,
    ),
    scratch_types=[
        pltpu.VMEM((_CROWS, _F), jnp.float32),
        pltpu.VMEM((_CROWS, _F), jnp.float32),
        pltpu.VMEM((_CROWS, _OF), jnp.float32),
        pltpu.VMEM((_CROWS, _OF), jnp.float32),
        pltpu.SemaphoreType.DMA,
        pltpu.SemaphoreType.DMA,
    ],
    compiler_params=pltpu.CompilerParams(needs_layout_passes=False),
)


def kernel(firings):
    return _sel(firings)


# speedup vs baseline: 1.3665x; 1.3665x over previous
import jax
import jax.numpy as jnp
from jax import lax
from jax.experimental import pallas as pl
from jax.experimental.pallas import tpu as pltpu
from jax.experimental.pallas import tpu_sc as plsc

_LANES = 16
_NUM_CORES = 2
_NUM_SUBCORES = 16
_NW = _NUM_CORES * _NUM_SUBCORES

_B, _R, _F = 4, 4096, 256
_OF = _F // 2
_ROWS = _B * _R
_ROWS_PER_W = _ROWS // _NW
_NCHUNK = 4
_CROWS = _ROWS_PER_W // _NCHUNK
_VPR = _OF // _LANES


def _sel_body(x_hbm, out_hbm, in_v0, in_v1, out_v0, out_v1, in_sem, out_sem):
    in_bufs = (in_v0, in_v1)
    out_bufs = (out_v0, out_v1)
    wid = lax.axis_index("s") * _NUM_CORES + lax.axis_index("c")
    b = wid // (_R // _ROWS_PER_W)
    r0 = (wid % (_R // _ROWS_PER_W)) * _ROWS_PER_W

    iota = lax.iota(jnp.int32, _LANES)
    cols = [iota * 2 + 32 * vj for vj in range(_VPR)]

    def start_in(t):
        return pltpu.async_copy(
            x_hbm.at[b, pl.ds(r0 + t * _CROWS, _CROWS), :],
            in_bufs[t % 2], in_sem)

    def start_out(t):
        return pltpu.async_copy(
            out_bufs[t % 2],
            out_hbm.at[b, pl.ds(r0 + t * _CROWS, _CROWS), :], out_sem)

    copies_in = [start_in(0)]
    copies_out = []
    for t in range(_NCHUNK):
        if t + 1 < _NCHUNK:
            copies_in.append(start_in(t + 1))
        copies_in[t].wait()
        if t >= 2:
            copies_out[t - 2].wait()
        src = in_bufs[t % 2]
        dst = out_bufs[t % 2]

        @plsc.parallel_loop(0, _CROWS, 1, unroll=4)
        def _gather_rows(dr):
            row = iota * 0 + dr
            for vj in range(_VPR):
                dst[dr, pl.ds(vj * _LANES, _LANES)] = (
                    plsc.load_gather(src, [row, cols[vj]]))
        copies_out.append(start_out(t))
    copies_out[-2].wait()
    copies_out[-1].wait()


_sel = pl.kernel(
    _sel_body,
    out_type=jax.ShapeDtypeStruct((_B, _R, _OF), jnp.float32),
    mesh=plsc.VectorSubcoreMesh(
        core_axis_name="c",
        subcore_axis_name="s",
        num_cores=_NUM_CORES,
        num_subcores=_NUM_SUBCORES,
    ),
    scratch_types=[
        pltpu.VMEM((_CROWS, _F), jnp.float32),
        pltpu.VMEM((_CROWS, _F), jnp.float32),
        pltpu.VMEM((_CROWS, _OF), jnp.float32),
        pltpu.VMEM((_CROWS, _OF), jnp.float32),
        pltpu.SemaphoreType.DMA,
        pltpu.SemaphoreType.DMA,
    ],
    compiler_params=pltpu.CompilerParams(needs_layout_passes=False),
)


def kernel(firings):
    return _sel(firings)


# speedup vs baseline: 1.3678x; 1.0010x over previous
import jax
import jax.numpy as jnp
from jax import lax
from jax.experimental import pallas as pl
from jax.experimental.pallas import tpu as pltpu
from jax.experimental.pallas import tpu_sc as plsc

_LANES = 16
_NUM_CORES = 2
_NUM_SUBCORES = 16
_NW = _NUM_CORES * _NUM_SUBCORES

_B, _R, _F = 4, 4096, 256
_OF = _F // 2
_ROWS = _B * _R
_ROWS_PER_W = _ROWS // _NW
_NCHUNK = 4
_CROWS = _ROWS_PER_W // _NCHUNK
_VPR = _OF // _LANES


def _sel_body(x_hbm, out_hbm, in_v0, in_v1, out_v0, out_v1, in_sem, out_sem):
    in_bufs = (in_v0, in_v1)
    out_bufs = (out_v0, out_v1)
    wid = lax.axis_index("s") * _NUM_CORES + lax.axis_index("c")
    b = wid // (_R // _ROWS_PER_W)
    r0 = (wid % (_R // _ROWS_PER_W)) * _ROWS_PER_W

    iota = lax.iota(jnp.int32, _LANES)
    cols = [iota * 2 + 32 * vj for vj in range(_VPR)]

    def start_in(t):
        return pltpu.async_copy(
            x_hbm.at[b, pl.ds(r0 + t * _CROWS, _CROWS), :],
            in_bufs[t % 2], in_sem)

    def start_out(t):
        return pltpu.async_copy(
            out_bufs[t % 2],
            out_hbm.at[b, pl.ds(r0 + t * _CROWS, _CROWS), :], out_sem)

    copies_in = [start_in(0)]
    copies_out = []
    for t in range(_NCHUNK):
        if t + 1 < _NCHUNK:
            copies_in.append(start_in(t + 1))
        copies_in[t].wait()
        if t >= 2:
            copies_out[t - 2].wait()
        src = in_bufs[t % 2]
        dst = out_bufs[t % 2]

        @plsc.parallel_loop(0, _CROWS, 1, unroll=4)
        def _gather_rows(dr):
            row = iota * 0 + dr
            for vj in range(_VPR):
                dst[dr, pl.ds(vj * _LANES, _LANES)] = (
                    plsc.load_gather(src, [row, cols[vj]]))
        copies_out.append(start_out(t))
    copies_out[-2].wait()
    copies_out[-1].wait()


_sel = pl.kernel(
    _sel_body,
    out_type=jax.ShapeDtypeStruct((_B, _R, _OF), jnp.float32),
    mesh=plsc.VectorSubcoreMesh(
        core_axis_name="c",
        subcore_axis_name="s",
        num_cores=_NUM_CORES,
        num_subcores=_NUM_SUBCORES,
    ),
    scratch_types=[
        pltpu.VMEM((_CROWS, _F), jnp.float32),
        pltpu.VMEM((_CROWS, _F), jnp.float32),
        pltpu.VMEM((_CROWS, _OF), jnp.float32),
        pltpu.VMEM((_CROWS, _OF), jnp.float32),
        pltpu.SemaphoreType.DMA,
        pltpu.SemaphoreType.DMA,
    ],
    compiler_params=pltpu.CompilerParams(
        needs_layout_passes=False,
        disable_bounds_checks=True,
        disable_semaphore_checks=True,
    ),
)


def kernel(firings):
    return _sel(firings)
